# Initial kernel scaffold; baseline (speedup 1.0000x reference)
import jax
import jax.numpy as jnp
from jax.experimental import pallas as pl


def kernel(x, edge_index, edge_attr, batch, W1, b1, W2, b2, p, Wl1, bl1, Wl2, bl2, Wl3, bl3):
    raise NotImplementedError("write your pallas kernel here")


# speedup vs baseline: 4.0688x; 4.0688x over previous
import functools

import jax
import jax.numpy as jnp
from jax import lax
from jax.experimental import pallas as pl
from jax.experimental.pallas import tpu as pltpu
from jax.experimental.pallas import tpu_sc as plsc

_N = 10000
_NP = 10240
_E = 160000
_EP = 163840
_H = 256
_HH = 128
_G = 64
_CH = 128
_NSUB = 16
_NCORE = 2
_STRIPE = _NP // _NSUB


def _splat(v16, t):
    idx = jnp.zeros((16,), jnp.int32) + t
    return v16.at[idx].get(mode="promise_in_bounds")


def _sc_mesh():
    return plsc.VectorSubcoreMesh(core_axis_name="c", subcore_axis_name="s")


def _sc_deg_call(dst, ew):
    ept = _EP // (_NCORE * _NSUB)
    nch = ept // _CH

    @functools.partial(
        pl.kernel,
        out_type=(
            jax.ShapeDtypeStruct((_NP, 16), jnp.float32),
            jax.ShapeDtypeStruct((_NP, 16), jnp.float32),
        ),
        mesh=_sc_mesh(),
        scratch_types=[
            pltpu.VMEM((_CH,), jnp.int32),
            pltpu.VMEM((_CH,), jnp.float32),
            pltpu.VMEM((_CH, 16), jnp.float32),
            pltpu.VMEM_SHARED((_NP, 16), jnp.float32),
        ],
    )
    def deg_k(dst_hbm, ew_hbm, out0, out1, dstv, ewv, rows, acc):
        c = lax.axis_index("c")
        s = lax.axis_index("s")
        wid = s * _NCORE + c

        def zero_rows(i, _):
            rows[i, :] = jnp.zeros((16,), jnp.float32)
            return 0

        lax.fori_loop(0, _CH, zero_rows, 0)
        for j in range(_STRIPE // _CH):
            pltpu.sync_copy(rows, acc.at[pl.ds(s * _STRIPE + j * _CH, _CH), :])
        plsc.subcore_barrier()

        def chunk(i, _):
            base = wid * ept + i * _CH
            pltpu.sync_copy(dst_hbm.at[pl.ds(base, _CH)], dstv)
            pltpu.sync_copy(ew_hbm.at[pl.ds(base, _CH)], ewv)

            for k in range(_CH // 16):
                wv = ewv[pl.ds(k * 16, 16)]

                def scale(t, _2, wv=wv, k=k):
                    w = _splat(wv, t)
                    v = rows[k * 16 + t, pl.ds(0, 16)]
                    rows[k * 16 + t, pl.ds(0, 16)] = v * 0.0 + w
                    return 0

                lax.fori_loop(0, 16, scale, 0)
            pltpu.sync_copy(rows, acc.at[dstv], add=True)
            return 0

        lax.fori_loop(0, nch, chunk, 0)
        plsc.subcore_barrier()

        @pl.when(c == 0)
        def _():
            pltpu.sync_copy(acc.at[pl.ds(s * _STRIPE, _STRIPE), :],
                            out0.at[pl.ds(s * _STRIPE, _STRIPE), :])

        @pl.when(c == 1)
        def _():
            pltpu.sync_copy(acc.at[pl.ds(s * _STRIPE, _STRIPE), :],
                            out1.at[pl.ds(s * _STRIPE, _STRIPE), :])

    return deg_k(dst, ew)


def _sc_agg_call(h0, h1, src, dst, ew):
    epc = _EP // _NSUB
    nch = epc // _CH

    @functools.partial(
        pl.kernel,
        out_type=(
            jax.ShapeDtypeStruct((_NP, _HH), jnp.float32),
            jax.ShapeDtypeStruct((_NP, _HH), jnp.float32),
        ),
        mesh=_sc_mesh(),
        scratch_types=[
            pltpu.VMEM((_CH,), jnp.int32),
            pltpu.VMEM((_CH,), jnp.int32),
            pltpu.VMEM((_CH,), jnp.float32),
            pltpu.VMEM((_CH, _HH), jnp.float32),
            pltpu.VMEM_SHARED((_NP, _HH), jnp.float32),
            pltpu.SemaphoreType.DMA,
        ],
    )
    def agg_k(h0_hbm, h1_hbm, src_hbm, dst_hbm, ew_hbm, out0, out1,
              srcv, dstv, ewv, rows, acc, sem):
        c = lax.axis_index("c")
        s = lax.axis_index("s")

        def zero_rows(i, _):
            for cc in range(_HH // 16):
                rows[i, pl.ds(cc * 16, 16)] = jnp.zeros((16,), jnp.float32)
            return 0

        lax.fori_loop(0, _CH, zero_rows, 0)
        for j in range(_STRIPE // _CH):
            pltpu.sync_copy(rows, acc.at[pl.ds(s * _STRIPE + j * _CH, _CH), :])
        plsc.subcore_barrier()

        def chunk(i, _):
            base = s * epc + i * _CH
            pltpu.sync_copy(src_hbm.at[pl.ds(base, _CH)], srcv)
            pltpu.sync_copy(dst_hbm.at[pl.ds(base, _CH)], dstv)
            pltpu.sync_copy(ew_hbm.at[pl.ds(base, _CH)], ewv)

            @pl.when(c == 0)
            def _():
                pltpu.async_copy(h0_hbm.at[srcv], rows, sem).wait()

            @pl.when(c == 1)
            def _():
                pltpu.async_copy(h1_hbm.at[srcv], rows, sem).wait()

            for k in range(_CH // 16):
                wv = ewv[pl.ds(k * 16, 16)]

                def scale(t, _2, wv=wv, k=k):
                    w = _splat(wv, t)
                    for cc in range(_HH // 16):
                        sl = pl.ds(cc * 16, 16)
                        rows[k * 16 + t, sl] = rows[k * 16 + t, sl] * w
                    return 0

                lax.fori_loop(0, 16, scale, 0)
            pltpu.sync_copy(rows, acc.at[dstv], add=True)
            return 0

        lax.fori_loop(0, nch, chunk, 0)
        plsc.subcore_barrier()

        @pl.when(c == 0)
        def _():
            pltpu.sync_copy(acc.at[pl.ds(s * _STRIPE, _STRIPE), :],
                            out0.at[pl.ds(s * _STRIPE, _STRIPE), :])

        @pl.when(c == 1)
        def _():
            pltpu.sync_copy(acc.at[pl.ds(s * _STRIPE, _STRIPE), :],
                            out1.at[pl.ds(s * _STRIPE, _STRIPE), :])

    return agg_k(h0, h1, src, dst, ew)


_RB = 256


def _k1_body(x_ref, w_ref, d0_ref, d1_ref, h0_ref, h1_ref, dinv_ref):
    deg = 1.0 + d0_ref[...] + d1_ref[...]
    dinv = lax.rsqrt(deg)
    hmat = jnp.dot(x_ref[...], w_ref[...], preferred_element_type=jnp.float32)
    ht = dinv * hmat
    h0_ref[...] = ht[:, :_HH]
    h1_ref[...] = ht[:, _HH:]
    dinv_ref[...] = dinv


def _k1_call(xp, w1, d0, d1):
    grid = (_NP // _RB,)
    return pl.pallas_call(
        _k1_body,
        grid=grid,
        in_specs=[
            pl.BlockSpec((_RB, 512), lambda i: (i, 0)),
            pl.BlockSpec((512, _H), lambda i: (0, 0)),
            pl.BlockSpec((_RB, 1), lambda i: (i, 0)),
            pl.BlockSpec((_RB, 1), lambda i: (i, 0)),
        ],
        out_specs=[
            pl.BlockSpec((_RB, _HH), lambda i: (i, 0)),
            pl.BlockSpec((_RB, _HH), lambda i: (i, 0)),
            pl.BlockSpec((_RB, 1), lambda i: (i, 0)),
        ],
        out_shape=[
            jax.ShapeDtypeStruct((_NP, _HH), jnp.float32),
            jax.ShapeDtypeStruct((_NP, _HH), jnp.float32),
            jax.ShapeDtypeStruct((_NP, 1), jnp.float32),
        ],
    )(xp, w1, d0, d1)


def _layer_body(a0_ref, a1_ref, h0_ref, h1_ref, dinv_ref, b_ref, w_ref,
                xk_ref, n0_ref, n1_ref):
    dinv = dinv_ref[...]
    agg = jnp.concatenate([a0_ref[...], a1_ref[...]], axis=1)
    htp = jnp.concatenate([h0_ref[...], h1_ref[...]], axis=1)
    xk = jnp.maximum(dinv * (agg + htp) + b_ref[...], 0.0)
    hn = jnp.dot(xk, w_ref[...], preferred_element_type=jnp.float32)
    htn = dinv * hn
    xk_ref[...] = xk
    n0_ref[...] = htn[:, :_HH]
    n1_ref[...] = htn[:, _HH:]


def _layer_call(a0, a1, h0, h1, dinv, brow, w2):
    grid = (_NP // _RB,)
    half = pl.BlockSpec((_RB, _HH), lambda i: (i, 0))
    return pl.pallas_call(
        _layer_body,
        grid=grid,
        in_specs=[
            half, half, half, half,
            pl.BlockSpec((_RB, 1), lambda i: (i, 0)),
            pl.BlockSpec((1, _H), lambda i: (0, 0)),
            pl.BlockSpec((_H, _H), lambda i: (0, 0)),
        ],
        out_specs=[pl.BlockSpec((_RB, _H), lambda i: (i, 0)), half, half],
        out_shape=[
            jax.ShapeDtypeStruct((_NP, _H), jnp.float32),
            jax.ShapeDtypeStruct((_NP, _HH), jnp.float32),
            jax.ShapeDtypeStruct((_NP, _HH), jnp.float32),
        ],
    )(a0, a1, h0, h1, dinv, brow, w2)


def _x3_body(a0_ref, a1_ref, h0_ref, h1_ref, dinv_ref, b_ref, x1_ref, x2_ref,
             p_ref, x3_ref, sc_ref):
    dinv = dinv_ref[...]
    agg = jnp.concatenate([a0_ref[...], a1_ref[...]], axis=1)
    htp = jnp.concatenate([h0_ref[...], h1_ref[...]], axis=1)
    x3 = jnp.maximum(dinv * (agg + htp) + b_ref[...], 0.0)
    cat = jnp.concatenate([x1_ref[...], x2_ref[...], x3], axis=1)
    p2 = p_ref[...]
    pn = jnp.sqrt(jnp.sum(p2 * p2))
    sco = jax.nn.sigmoid(
        jnp.dot(cat, p2, preferred_element_type=jnp.float32) / pn)
    x3_ref[...] = x3
    sc_ref[...] = sco


def _x3_call(a0, a1, h0, h1, dinv, brow, x1, x2, pcol):
    grid = (_NP // _RB,)
    half = pl.BlockSpec((_RB, _HH), lambda i: (i, 0))
    return pl.pallas_call(
        _x3_body,
        grid=grid,
        in_specs=[
            half, half, half, half,
            pl.BlockSpec((_RB, 1), lambda i: (i, 0)),
            pl.BlockSpec((1, _H), lambda i: (0, 0)),
            pl.BlockSpec((_RB, _H), lambda i: (i, 0)),
            pl.BlockSpec((_RB, _H), lambda i: (i, 0)),
            pl.BlockSpec((3 * _H, 1), lambda i: (0, 0)),
        ],
        out_specs=[
            pl.BlockSpec((_RB, _H), lambda i: (i, 0)),
            pl.BlockSpec((_RB, 1), lambda i: (i, 0)),
        ],
        out_shape=[
            jax.ShapeDtypeStruct((_NP, _H), jnp.float32),
            jax.ShapeDtypeStruct((_NP, 1), jnp.float32),
        ],
    )(a0, a1, h0, h1, dinv, brow, x1, x2, pcol)


_JC = 1024


def _rank_body(srow_ref, brow_ref, scol_ref, bcol_ref, keep_ref, cnt_ref):
    pid = pl.program_id(0)
    srow = srow_ref[...]
    brow = brow_ref[...]
    si = scol_ref[...]
    bi = bcol_ref[...]
    ii = lax.broadcasted_iota(jnp.int32, (_RB, 1), 0) + pid * _RB
    gmin = jnp.min(bi)
    gmax = jnp.max(bi)

    cnt_ref[...] = jnp.zeros((_RB, 1), jnp.float32)
    cg = jnp.zeros((_G, 1), jnp.float32)
    gcol = lax.broadcasted_iota(jnp.int32, (_G, 1), 0)
    for jc in range(_NP // _JC):
        bj = lax.slice(brow, (0, jc * _JC), (1, (jc + 1) * _JC))
        cg = cg + jnp.sum((bj == gcol).astype(jnp.float32), axis=1,
                          keepdims=True)
        sj = lax.slice(srow, (0, jc * _JC), (1, (jc + 1) * _JC))
        ij = lax.broadcasted_iota(jnp.int32, (1, _JC), 1) + jc * _JC
        cmin = jnp.min(bj)
        cmax = jnp.max(bj)

        @pl.when((cmax >= gmin) & (cmin <= gmax))
        def _(bj=bj, sj=sj, ij=ij):
            same = bj == bi
            gt = (sj > si) | ((sj == si) & (ij < ii))
            cnt_ref[...] += jnp.sum((same & gt).astype(jnp.float32), axis=1,
                                    keepdims=True)

    kvec = jnp.ceil(0.8 * cg)
    oh = (bi == gcol.reshape(1, _G)).astype(jnp.float32)
    kati = jnp.dot(oh, kvec, preferred_element_type=jnp.float32)
    keep_ref[...] = (cnt_ref[...] < kati).astype(jnp.float32)


def _rank_call(srow, brow, scol, bcol):
    grid = (_NP // _RB,)
    return pl.pallas_call(
        _rank_body,
        grid=grid,
        in_specs=[
            pl.BlockSpec((1, _NP), lambda i: (0, 0)),
            pl.BlockSpec((1, _NP), lambda i: (0, 0)),
            pl.BlockSpec((_RB, 1), lambda i: (i, 0)),
            pl.BlockSpec((_RB, 1), lambda i: (i, 0)),
        ],
        out_specs=pl.BlockSpec((_RB, 1), lambda i: (i, 0)),
        out_shape=jax.ShapeDtypeStruct((_NP, 1), jnp.float32),
        scratch_shapes=[pltpu.VMEM((_RB, 1), jnp.float32)],
    )(srow, brow, scol, bcol)


_PB = 1024


def _pool_body(x1_ref, x2_ref, x3_ref, sc_ref, kp_ref, bc_ref,
               sum_ref, max_ref, cnt_ref):
    pid = pl.program_id(0)

    @pl.when(pid == 0)
    def _():
        sum_ref[...] = jnp.zeros((_G, 3 * _H), jnp.float32)
        max_ref[...] = jnp.full((_G, 3 * _H), -jnp.inf, jnp.float32)
        cnt_ref[...] = jnp.zeros((_G, 1), jnp.float32)

    cat = jnp.concatenate([x1_ref[...], x2_ref[...], x3_ref[...]], axis=1)
    xp = cat * sc_ref[...]
    kf = kp_ref[...]
    b = bc_ref[...]
    grow = lax.broadcasted_iota(jnp.int32, (1, _G), 1)
    oh = ((b == grow) & (kf > 0.5)).astype(jnp.float32)
    sum_ref[...] += lax.dot_general(oh, xp, (((0,), (0,)), ((), ())),
                                    preferred_element_type=jnp.float32)
    cnt_ref[...] += jnp.sum(oh, axis=0)[:, None]
    gmn = jnp.min(b)
    gmx = jnp.max(b)
    for g in range(_G):
        @pl.when((g >= gmn) & (g <= gmx))
        def _(g=g):
            m = (b == g) & (kf > 0.5)
            mx = jnp.max(jnp.where(m, xp, -jnp.inf), axis=0)
            max_ref[g, :] = jnp.maximum(max_ref[g, :], mx)


def _pool_call(x1, x2, x3, score, keep, bcol):
    grid = (_NP // _PB,)
    full = pl.BlockSpec((_PB, _H), lambda i: (i, 0))
    one = pl.BlockSpec((_PB, 1), lambda i: (i, 0))
    return pl.pallas_call(
        _pool_body,
        grid=grid,
        in_specs=[full, full, full, one, one, one],
        out_specs=[
            pl.BlockSpec((_G, 3 * _H), lambda i: (0, 0)),
            pl.BlockSpec((_G, 3 * _H), lambda i: (0, 0)),
            pl.BlockSpec((_G, 1), lambda i: (0, 0)),
        ],
        out_shape=[
            jax.ShapeDtypeStruct((_G, 3 * _H), jnp.float32),
            jax.ShapeDtypeStruct((_G, 3 * _H), jnp.float32),
            jax.ShapeDtypeStruct((_G, 1), jnp.float32),
        ],
    )(x1, x2, x3, score, keep, bcol)


def _mlp_body(sum_ref, max_ref, cnt_ref, w1_ref, b1_ref, w2_ref, b2_ref,
              w3_ref, b3_ref, out_ref):
    cnt = cnt_ref[...]
    gap = sum_ref[...] / jnp.maximum(cnt, 1.0)
    mx = max_ref[...]
    gmp = jnp.where(jnp.isfinite(mx), mx, 0.0)
    ro = jnp.concatenate([gap, gmp], axis=1)
    h = jnp.maximum(
        jnp.dot(ro, w1_ref[...], preferred_element_type=jnp.float32)
        + b1_ref[...], 0.0)
    h = jnp.maximum(
        jnp.dot(h, w2_ref[...], preferred_element_type=jnp.float32)
        + b2_ref[...], 0.0)
    out_ref[...] = (
        jnp.dot(h, w3_ref[...], preferred_element_type=jnp.float32)
        + b3_ref[...])


def _mlp_call(sums, maxs, cnts, wl1, bl1, wl2, bl2, wl3, bl3):
    return pl.pallas_call(
        _mlp_body,
        out_shape=jax.ShapeDtypeStruct((_G, 3), jnp.float32),
    )(sums, maxs, cnts, wl1, bl1.reshape(1, -1), wl2, bl2.reshape(1, -1),
      wl3, bl3.reshape(1, -1))


def kernel(x, edge_index, edge_attr, batch, W1, b1, W2, b2, p,
           Wl1, bl1, Wl2, bl2, Wl3, bl3):
    src = edge_index[0].astype(jnp.int32)
    dst = edge_index[1].astype(jnp.int32)
    ew = edge_attr.astype(jnp.float32)
    pad_e = _EP - _E
    src = jnp.pad(src, (0, pad_e))
    dst = jnp.pad(dst, (0, pad_e))
    ew = jnp.pad(ew, (0, pad_e))

    xp = jnp.pad(x, ((0, _NP - _N), (0, 0)))
    bcol = jnp.pad(batch.astype(jnp.int32), (0, _NP - _N),
                   constant_values=_G).reshape(_NP, 1)

    dp0, dp1 = _sc_deg_call(dst, ew)
    d0 = dp0[:, 0:1]
    d1 = dp1[:, 0:1]

    h0, h1, dinv = _k1_call(xp, W1, d0, d1)

    b1row = b1.reshape(1, _H)
    b2row = b2.reshape(1, _H)

    a0, a1 = _sc_agg_call(h0, h1, src, dst, ew)
    x1, h0, h1 = _layer_call(a0, a1, h0, h1, dinv, b1row, W2)
    a0, a1 = _sc_agg_call(h0, h1, src, dst, ew)
    x2, h0, h1 = _layer_call(a0, a1, h0, h1, dinv, b2row, W2)
    a0, a1 = _sc_agg_call(h0, h1, src, dst, ew)
    x3, score = _x3_call(a0, a1, h0, h1, dinv, b2row, x1, x2,
                         p.reshape(3 * _H, 1))

    srow = score.reshape(1, _NP)
    brow = bcol.reshape(1, _NP)
    keep = _rank_call(srow, brow, score, bcol)

    sums, maxs, cnts = _pool_call(x1, x2, x3, score, keep, bcol)
    return _mlp_call(sums, maxs, cnts, Wl1, bl1, Wl2, bl2, Wl3, bl3)


# speedup vs baseline: 5.1526x; 1.2664x over previous
import functools

import jax
import jax.numpy as jnp
from jax import lax
from jax.experimental import pallas as pl
from jax.experimental.pallas import tpu as pltpu
from jax.experimental.pallas import tpu_sc as plsc

_N = 10000
_NP = 10240
_E = 160000
_EP = 163840
_H = 256
_HH = 128
_G = 64
_CH = 128
_NSUB = 16
_NCORE = 2
_STRIPE = _NP // _NSUB


def _splat(v16, t):
    idx = jnp.zeros((16,), jnp.int32) + t
    return v16.at[idx].get(mode="promise_in_bounds")


def _sc_mesh():
    return plsc.VectorSubcoreMesh(core_axis_name="c", subcore_axis_name="s")


def _sc_deg_call(dst, ew):
    ept = _EP // (_NCORE * _NSUB)
    nch = ept // _CH

    @functools.partial(
        pl.kernel,
        out_type=(
            jax.ShapeDtypeStruct((_NP, 16), jnp.float32),
            jax.ShapeDtypeStruct((_NP, 16), jnp.float32),
        ),
        mesh=_sc_mesh(),
        scratch_types=[
            pltpu.VMEM((_CH,), jnp.int32),
            pltpu.VMEM((_CH,), jnp.float32),
            pltpu.VMEM((_CH, 16), jnp.float32),
            pltpu.VMEM_SHARED((_NP, 16), jnp.float32),
        ],
    )
    def deg_k(dst_hbm, ew_hbm, out0, out1, dstv, ewv, rows, acc):
        c = lax.axis_index("c")
        s = lax.axis_index("s")
        wid = s * _NCORE + c

        def zero_rows(i, _):
            rows[i, :] = jnp.zeros((16,), jnp.float32)
            return 0

        lax.fori_loop(0, _CH, zero_rows, 0)
        for j in range(_STRIPE // _CH):
            pltpu.sync_copy(rows, acc.at[pl.ds(s * _STRIPE + j * _CH, _CH), :])
        plsc.subcore_barrier()

        def chunk(i, _):
            base = wid * ept + i * _CH
            pltpu.sync_copy(dst_hbm.at[pl.ds(base, _CH)], dstv)
            pltpu.sync_copy(ew_hbm.at[pl.ds(base, _CH)], ewv)

            for k in range(_CH // 16):
                wv = ewv[pl.ds(k * 16, 16)]

                def scale(t, _2, wv=wv, k=k):
                    w = _splat(wv, t)
                    v = rows[k * 16 + t, pl.ds(0, 16)]
                    rows[k * 16 + t, pl.ds(0, 16)] = v * 0.0 + w
                    return 0

                lax.fori_loop(0, 16, scale, 0)
            pltpu.sync_copy(rows, acc.at[dstv], add=True)
            return 0

        lax.fori_loop(0, nch, chunk, 0)
        plsc.subcore_barrier()

        @pl.when(c == 0)
        def _():
            pltpu.sync_copy(acc.at[pl.ds(s * _STRIPE, _STRIPE), :],
                            out0.at[pl.ds(s * _STRIPE, _STRIPE), :])

        @pl.when(c == 1)
        def _():
            pltpu.sync_copy(acc.at[pl.ds(s * _STRIPE, _STRIPE), :],
                            out1.at[pl.ds(s * _STRIPE, _STRIPE), :])

    return deg_k(dst, ew)


def _sc_agg_call(h0, h1, src, dst, ew):
    epc = _EP // _NSUB
    nch = epc // _CH
    nit = nch // 2

    @functools.partial(
        pl.kernel,
        out_type=(
            jax.ShapeDtypeStruct((_NP, _HH), jnp.float32),
            jax.ShapeDtypeStruct((_NP, _HH), jnp.float32),
        ),
        mesh=_sc_mesh(),
        scratch_types=[
            pltpu.VMEM((_CH,), jnp.int32),
            pltpu.VMEM((_CH,), jnp.int32),
            pltpu.VMEM((_CH,), jnp.float32),
            pltpu.VMEM((_CH, _HH), jnp.float32),
            pltpu.VMEM((_CH,), jnp.int32),
            pltpu.VMEM((_CH,), jnp.int32),
            pltpu.VMEM((_CH,), jnp.float32),
            pltpu.VMEM((_CH, _HH), jnp.float32),
            pltpu.VMEM_SHARED((_NP, _HH), jnp.float32),
            pltpu.SemaphoreType.DMA,
            pltpu.SemaphoreType.DMA,
        ],
    )
    def agg_k(h0_hbm, h1_hbm, src_hbm, dst_hbm, ew_hbm, out0, out1,
              srcv0, dstv0, ewv0, rows0, srcv1, dstv1, ewv1, rows1,
              acc, gs0, gs1):
        c = lax.axis_index("c")
        s = lax.axis_index("s")
        bufs = ((srcv0, dstv0, ewv0, rows0, gs0),
                (srcv1, dstv1, ewv1, rows1, gs1))

        def zero_rows(i, _):
            for cc in range(_HH // 16):
                rows0[i, pl.ds(cc * 16, 16)] = jnp.zeros((16,), jnp.float32)
            return 0

        lax.fori_loop(0, _CH, zero_rows, 0)
        for j in range(_STRIPE // _CH):
            pltpu.sync_copy(rows0, acc.at[pl.ds(s * _STRIPE + j * _CH, _CH), :])
        plsc.subcore_barrier()

        def load_idx(b, ci):
            sv, dv, wv, _, _ = bufs[b]
            base = s * epc + ci * _CH
            pltpu.sync_copy(src_hbm.at[pl.ds(base, _CH)], sv)
            pltpu.sync_copy(dst_hbm.at[pl.ds(base, _CH)], dv)
            pltpu.sync_copy(ew_hbm.at[pl.ds(base, _CH)], wv)

        def fire_gather(b):
            sv, _, _, rw, sem = bufs[b]

            @pl.when(c == 0)
            def _():
                pltpu.async_copy(h0_hbm.at[sv], rw, sem)

            @pl.when(c == 1)
            def _():
                pltpu.async_copy(h1_hbm.at[sv], rw, sem)

        def wait_gather(b):
            sv, _, _, rw, sem = bufs[b]

            @pl.when(c == 0)
            def _():
                pltpu.make_async_copy(h0_hbm.at[sv], rw, sem).wait()

            @pl.when(c == 1)
            def _():
                pltpu.make_async_copy(h1_hbm.at[sv], rw, sem).wait()

        def scale_scatter(b):
            _, dv, wv, rw, _ = bufs[b]
            for k in range(_CH // 16):
                wvec = wv[pl.ds(k * 16, 16)]

                def scale(t, _2, wvec=wvec, k=k):
                    w = _splat(wvec, t)
                    for cc in range(_HH // 16):
                        sl = pl.ds(cc * 16, 16)
                        rw[k * 16 + t, sl] = rw[k * 16 + t, sl] * w
                    return 0

                lax.fori_loop(0, 16, scale, 0)
            pltpu.sync_copy(rw, acc.at[dv], add=True)

        load_idx(0, 0)
        fire_gather(0)

        def itbody(j, _):
            i0 = 2 * j
            load_idx(1, i0 + 1)
            fire_gather(1)
            wait_gather(0)
            scale_scatter(0)

            @pl.when(j + 1 < nit)
            def _():
                load_idx(0, i0 + 2)
                fire_gather(0)

            wait_gather(1)
            scale_scatter(1)
            return 0

        lax.fori_loop(0, nit, itbody, 0)
        plsc.subcore_barrier()

        @pl.when(c == 0)
        def _():
            pltpu.sync_copy(acc.at[pl.ds(s * _STRIPE, _STRIPE), :],
                            out0.at[pl.ds(s * _STRIPE, _STRIPE), :])

        @pl.when(c == 1)
        def _():
            pltpu.sync_copy(acc.at[pl.ds(s * _STRIPE, _STRIPE), :],
                            out1.at[pl.ds(s * _STRIPE, _STRIPE), :])

    return agg_k(h0, h1, src, dst, ew)


_RB = 256


def _k1_body(x_ref, w_ref, d0_ref, d1_ref, h0_ref, h1_ref, dinv_ref):
    deg = 1.0 + d0_ref[...] + d1_ref[...]
    dinv = lax.rsqrt(deg)
    hmat = jnp.dot(x_ref[...], w_ref[...], preferred_element_type=jnp.float32)
    ht = dinv * hmat
    h0_ref[...] = ht[:, :_HH]
    h1_ref[...] = ht[:, _HH:]
    dinv_ref[...] = dinv


def _k1_call(xp, w1, d0, d1):
    grid = (_NP // _RB,)
    return pl.pallas_call(
        _k1_body,
        grid=grid,
        in_specs=[
            pl.BlockSpec((_RB, 512), lambda i: (i, 0)),
            pl.BlockSpec((512, _H), lambda i: (0, 0)),
            pl.BlockSpec((_RB, 1), lambda i: (i, 0)),
            pl.BlockSpec((_RB, 1), lambda i: (i, 0)),
        ],
        out_specs=[
            pl.BlockSpec((_RB, _HH), lambda i: (i, 0)),
            pl.BlockSpec((_RB, _HH), lambda i: (i, 0)),
            pl.BlockSpec((_RB, 1), lambda i: (i, 0)),
        ],
        out_shape=[
            jax.ShapeDtypeStruct((_NP, _HH), jnp.float32),
            jax.ShapeDtypeStruct((_NP, _HH), jnp.float32),
            jax.ShapeDtypeStruct((_NP, 1), jnp.float32),
        ],
    )(xp, w1, d0, d1)


def _layer_body(a0_ref, a1_ref, h0_ref, h1_ref, dinv_ref, b_ref, w_ref,
                xk_ref, n0_ref, n1_ref):
    dinv = dinv_ref[...]
    agg = jnp.concatenate([a0_ref[...], a1_ref[...]], axis=1)
    htp = jnp.concatenate([h0_ref[...], h1_ref[...]], axis=1)
    xk = jnp.maximum(dinv * (agg + htp) + b_ref[...], 0.0)
    hn = jnp.dot(xk, w_ref[...], preferred_element_type=jnp.float32)
    htn = dinv * hn
    xk_ref[...] = xk
    n0_ref[...] = htn[:, :_HH]
    n1_ref[...] = htn[:, _HH:]


def _layer_call(a0, a1, h0, h1, dinv, brow, w2):
    grid = (_NP // _RB,)
    half = pl.BlockSpec((_RB, _HH), lambda i: (i, 0))
    return pl.pallas_call(
        _layer_body,
        grid=grid,
        in_specs=[
            half, half, half, half,
            pl.BlockSpec((_RB, 1), lambda i: (i, 0)),
            pl.BlockSpec((1, _H), lambda i: (0, 0)),
            pl.BlockSpec((_H, _H), lambda i: (0, 0)),
        ],
        out_specs=[pl.BlockSpec((_RB, _H), lambda i: (i, 0)), half, half],
        out_shape=[
            jax.ShapeDtypeStruct((_NP, _H), jnp.float32),
            jax.ShapeDtypeStruct((_NP, _HH), jnp.float32),
            jax.ShapeDtypeStruct((_NP, _HH), jnp.float32),
        ],
    )(a0, a1, h0, h1, dinv, brow, w2)


def _x3_body(a0_ref, a1_ref, h0_ref, h1_ref, dinv_ref, b_ref, x1_ref, x2_ref,
             p_ref, x3_ref, sc_ref):
    dinv = dinv_ref[...]
    agg = jnp.concatenate([a0_ref[...], a1_ref[...]], axis=1)
    htp = jnp.concatenate([h0_ref[...], h1_ref[...]], axis=1)
    x3 = jnp.maximum(dinv * (agg + htp) + b_ref[...], 0.0)
    cat = jnp.concatenate([x1_ref[...], x2_ref[...], x3], axis=1)
    p2 = p_ref[...]
    pn = jnp.sqrt(jnp.sum(p2 * p2))
    sco = jax.nn.sigmoid(
        jnp.dot(cat, p2, preferred_element_type=jnp.float32) / pn)
    x3_ref[...] = x3
    sc_ref[...] = sco


def _x3_call(a0, a1, h0, h1, dinv, brow, x1, x2, pcol):
    grid = (_NP // _RB,)
    half = pl.BlockSpec((_RB, _HH), lambda i: (i, 0))
    return pl.pallas_call(
        _x3_body,
        grid=grid,
        in_specs=[
            half, half, half, half,
            pl.BlockSpec((_RB, 1), lambda i: (i, 0)),
            pl.BlockSpec((1, _H), lambda i: (0, 0)),
            pl.BlockSpec((_RB, _H), lambda i: (i, 0)),
            pl.BlockSpec((_RB, _H), lambda i: (i, 0)),
            pl.BlockSpec((3 * _H, 1), lambda i: (0, 0)),
        ],
        out_specs=[
            pl.BlockSpec((_RB, _H), lambda i: (i, 0)),
            pl.BlockSpec((_RB, 1), lambda i: (i, 0)),
        ],
        out_shape=[
            jax.ShapeDtypeStruct((_NP, _H), jnp.float32),
            jax.ShapeDtypeStruct((_NP, 1), jnp.float32),
        ],
    )(a0, a1, h0, h1, dinv, brow, x1, x2, pcol)


_JC = 1024


def _rank_body(srow_ref, brow_ref, scol_ref, bcol_ref, keep_ref, cnt_ref):
    pid = pl.program_id(0)
    srow = srow_ref[...]
    brow = brow_ref[...]
    si = scol_ref[...]
    bi = bcol_ref[...]
    ii = lax.broadcasted_iota(jnp.int32, (_RB, 1), 0) + pid * _RB
    gmin = jnp.min(bi)
    gmax = jnp.max(bi)

    cnt_ref[...] = jnp.zeros((_RB, 1), jnp.float32)
    cg = jnp.zeros((_G, 1), jnp.float32)
    gcol = lax.broadcasted_iota(jnp.int32, (_G, 1), 0)
    for jc in range(_NP // _JC):
        bj = lax.slice(brow, (0, jc * _JC), (1, (jc + 1) * _JC))
        cg = cg + jnp.sum((bj == gcol).astype(jnp.float32), axis=1,
                          keepdims=True)
        sj = lax.slice(srow, (0, jc * _JC), (1, (jc + 1) * _JC))
        ij = lax.broadcasted_iota(jnp.int32, (1, _JC), 1) + jc * _JC
        cmin = jnp.min(bj)
        cmax = jnp.max(bj)

        @pl.when((cmax >= gmin) & (cmin <= gmax))
        def _(bj=bj, sj=sj, ij=ij):
            same = bj == bi
            gt = (sj > si) | ((sj == si) & (ij < ii))
            cnt_ref[...] += jnp.sum((same & gt).astype(jnp.float32), axis=1,
                                    keepdims=True)

    kvec = jnp.ceil(0.8 * cg)
    oh = (bi == gcol.reshape(1, _G)).astype(jnp.float32)
    kati = jnp.dot(oh, kvec, preferred_element_type=jnp.float32)
    keep_ref[...] = (cnt_ref[...] < kati).astype(jnp.float32)


def _rank_call(srow, brow, scol, bcol):
    grid = (_NP // _RB,)
    return pl.pallas_call(
        _rank_body,
        grid=grid,
        in_specs=[
            pl.BlockSpec((1, _NP), lambda i: (0, 0)),
            pl.BlockSpec((1, _NP), lambda i: (0, 0)),
            pl.BlockSpec((_RB, 1), lambda i: (i, 0)),
            pl.BlockSpec((_RB, 1), lambda i: (i, 0)),
        ],
        out_specs=pl.BlockSpec((_RB, 1), lambda i: (i, 0)),
        out_shape=jax.ShapeDtypeStruct((_NP, 1), jnp.float32),
        scratch_shapes=[pltpu.VMEM((_RB, 1), jnp.float32)],
    )(srow, brow, scol, bcol)


_PB = 1024


def _pool_body(x1_ref, x2_ref, x3_ref, sc_ref, kp_ref, bc_ref,
               sum_ref, max_ref, cnt_ref):
    pid = pl.program_id(0)

    @pl.when(pid == 0)
    def _():
        sum_ref[...] = jnp.zeros((_G, 3 * _H), jnp.float32)
        max_ref[...] = jnp.full((_G, 3 * _H), -jnp.inf, jnp.float32)
        cnt_ref[...] = jnp.zeros((_G, 1), jnp.float32)

    cat = jnp.concatenate([x1_ref[...], x2_ref[...], x3_ref[...]], axis=1)
    xp = cat * sc_ref[...]
    kf = kp_ref[...]
    b = bc_ref[...]
    grow = lax.broadcasted_iota(jnp.int32, (1, _G), 1)
    oh = ((b == grow) & (kf > 0.5)).astype(jnp.float32)
    sum_ref[...] += lax.dot_general(oh, xp, (((0,), (0,)), ((), ())),
                                    preferred_element_type=jnp.float32)
    cnt_ref[...] += jnp.sum(oh, axis=0)[:, None]
    gmn = jnp.min(b)
    gmx = jnp.max(b)
    for g in range(_G):
        @pl.when((g >= gmn) & (g <= gmx))
        def _(g=g):
            m = (b == g) & (kf > 0.5)
            mx = jnp.max(jnp.where(m, xp, -jnp.inf), axis=0)
            max_ref[g, :] = jnp.maximum(max_ref[g, :], mx)


def _pool_call(x1, x2, x3, score, keep, bcol):
    grid = (_NP // _PB,)
    full = pl.BlockSpec((_PB, _H), lambda i: (i, 0))
    one = pl.BlockSpec((_PB, 1), lambda i: (i, 0))
    return pl.pallas_call(
        _pool_body,
        grid=grid,
        in_specs=[full, full, full, one, one, one],
        out_specs=[
            pl.BlockSpec((_G, 3 * _H), lambda i: (0, 0)),
            pl.BlockSpec((_G, 3 * _H), lambda i: (0, 0)),
            pl.BlockSpec((_G, 1), lambda i: (0, 0)),
        ],
        out_shape=[
            jax.ShapeDtypeStruct((_G, 3 * _H), jnp.float32),
            jax.ShapeDtypeStruct((_G, 3 * _H), jnp.float32),
            jax.ShapeDtypeStruct((_G, 1), jnp.float32),
        ],
    )(x1, x2, x3, score, keep, bcol)


def _mlp_body(sum_ref, max_ref, cnt_ref, w1_ref, b1_ref, w2_ref, b2_ref,
              w3_ref, b3_ref, out_ref):
    cnt = cnt_ref[...]
    gap = sum_ref[...] / jnp.maximum(cnt, 1.0)
    mx = max_ref[...]
    gmp = jnp.where(jnp.isfinite(mx), mx, 0.0)
    ro = jnp.concatenate([gap, gmp], axis=1)
    h = jnp.maximum(
        jnp.dot(ro, w1_ref[...], preferred_element_type=jnp.float32)
        + b1_ref[...], 0.0)
    h = jnp.maximum(
        jnp.dot(h, w2_ref[...], preferred_element_type=jnp.float32)
        + b2_ref[...], 0.0)
    out_ref[...] = (
        jnp.dot(h, w3_ref[...], preferred_element_type=jnp.float32)
        + b3_ref[...])


def _mlp_call(sums, maxs, cnts, wl1, bl1, wl2, bl2, wl3, bl3):
    return pl.pallas_call(
        _mlp_body,
        out_shape=jax.ShapeDtypeStruct((_G, 3), jnp.float32),
    )(sums, maxs, cnts, wl1, bl1.reshape(1, -1), wl2, bl2.reshape(1, -1),
      wl3, bl3.reshape(1, -1))


def kernel(x, edge_index, edge_attr, batch, W1, b1, W2, b2, p,
           Wl1, bl1, Wl2, bl2, Wl3, bl3):
    src = edge_index[0].astype(jnp.int32)
    dst = edge_index[1].astype(jnp.int32)
    ew = edge_attr.astype(jnp.float32)
    pad_e = _EP - _E
    src = jnp.pad(src, (0, pad_e))
    dst = jnp.pad(dst, (0, pad_e))
    ew = jnp.pad(ew, (0, pad_e))

    xp = jnp.pad(x, ((0, _NP - _N), (0, 0)))
    bcol = jnp.pad(batch.astype(jnp.int32), (0, _NP - _N),
                   constant_values=_G).reshape(_NP, 1)

    dp0, dp1 = _sc_deg_call(dst, ew)
    d0 = dp0[:, 0:1]
    d1 = dp1[:, 0:1]

    h0, h1, dinv = _k1_call(xp, W1, d0, d1)

    b1row = b1.reshape(1, _H)
    b2row = b2.reshape(1, _H)

    a0, a1 = _sc_agg_call(h0, h1, src, dst, ew)
    x1, h0, h1 = _layer_call(a0, a1, h0, h1, dinv, b1row, W2)
    a0, a1 = _sc_agg_call(h0, h1, src, dst, ew)
    x2, h0, h1 = _layer_call(a0, a1, h0, h1, dinv, b2row, W2)
    a0, a1 = _sc_agg_call(h0, h1, src, dst, ew)
    x3, score = _x3_call(a0, a1, h0, h1, dinv, b2row, x1, x2,
                         p.reshape(3 * _H, 1))

    srow = score.reshape(1, _NP)
    brow = bcol.reshape(1, _NP)
    keep = _rank_call(srow, brow, score, bcol)

    sums, maxs, cnts = _pool_call(x1, x2, x3, score, keep, bcol)
    return _mlp_call(sums, maxs, cnts, Wl1, bl1, Wl2, bl2, Wl3, bl3)


# speedup vs baseline: 5.7650x; 1.1189x over previous
import functools

import jax
import jax.numpy as jnp
from jax import lax
from jax.experimental import pallas as pl
from jax.experimental.pallas import tpu as pltpu
from jax.experimental.pallas import tpu_sc as plsc

_N = 10000
_NP = 10240
_E = 160000
_EP = 163840
_H = 256
_HH = 128
_G = 64
_CH = 128
_NSUB = 16
_NCORE = 2
_STRIPE = _NP // _NSUB


def _splat(v16, t):
    idx = jnp.zeros((16,), jnp.int32) + t
    return v16.at[idx].get(mode="promise_in_bounds")


def _sc_mesh():
    return plsc.VectorSubcoreMesh(core_axis_name="c", subcore_axis_name="s")


def _sc_deg_call(dst, ew):
    ept = _EP // (_NCORE * _NSUB)
    nch = ept // _CH

    @functools.partial(
        pl.kernel,
        out_type=(
            jax.ShapeDtypeStruct((_NP, 16), jnp.float32),
            jax.ShapeDtypeStruct((_NP, 16), jnp.float32),
        ),
        mesh=_sc_mesh(),
        scratch_types=[
            pltpu.VMEM((_CH,), jnp.int32),
            pltpu.VMEM((_CH,), jnp.float32),
            pltpu.VMEM((_CH, 16), jnp.float32),
            pltpu.VMEM_SHARED((_NP, 16), jnp.float32),
        ],
    )
    def deg_k(dst_hbm, ew_hbm, out0, out1, dstv, ewv, rows, acc):
        c = lax.axis_index("c")
        s = lax.axis_index("s")
        wid = s * _NCORE + c

        def zero_rows(i, _):
            rows[i, :] = jnp.zeros((16,), jnp.float32)
            return 0

        lax.fori_loop(0, _CH, zero_rows, 0)
        for j in range(_STRIPE // _CH):
            pltpu.sync_copy(rows, acc.at[pl.ds(s * _STRIPE + j * _CH, _CH), :])
        plsc.subcore_barrier()

        def chunk(i, _):
            base = wid * ept + i * _CH
            pltpu.sync_copy(dst_hbm.at[pl.ds(base, _CH)], dstv)
            pltpu.sync_copy(ew_hbm.at[pl.ds(base, _CH)], ewv)

            for k in range(_CH // 16):
                wv = ewv[pl.ds(k * 16, 16)]

                def scale(t, _2, wv=wv, k=k):
                    w = _splat(wv, t)
                    v = rows[k * 16 + t, pl.ds(0, 16)]
                    rows[k * 16 + t, pl.ds(0, 16)] = v * 0.0 + w
                    return 0

                lax.fori_loop(0, 16, scale, 0)
            pltpu.sync_copy(rows, acc.at[dstv], add=True)
            return 0

        lax.fori_loop(0, nch, chunk, 0)
        plsc.subcore_barrier()

        @pl.when(c == 0)
        def _():
            pltpu.sync_copy(acc.at[pl.ds(s * _STRIPE, _STRIPE), :],
                            out0.at[pl.ds(s * _STRIPE, _STRIPE), :])

        @pl.when(c == 1)
        def _():
            pltpu.sync_copy(acc.at[pl.ds(s * _STRIPE, _STRIPE), :],
                            out1.at[pl.ds(s * _STRIPE, _STRIPE), :])

    return deg_k(dst, ew)


def _sc_agg_call(h0, h1, src, dst, ew):
    epc = _EP // _NSUB
    nch = epc // _CH
    nit = nch // 2

    @functools.partial(
        pl.kernel,
        out_type=(
            jax.ShapeDtypeStruct((_NP, _HH), jnp.float32),
            jax.ShapeDtypeStruct((_NP, _HH), jnp.float32),
        ),
        mesh=_sc_mesh(),
        scratch_types=[
            pltpu.VMEM((_EP // _NSUB,), jnp.int32),
            pltpu.VMEM((_CH,), jnp.int32),
            pltpu.VMEM((_CH,), jnp.float32),
            pltpu.VMEM((_CH, _HH), jnp.float32),
            pltpu.VMEM((_CH,), jnp.int32),
            pltpu.VMEM((_CH,), jnp.float32),
            pltpu.VMEM((_CH, _HH), jnp.float32),
            pltpu.VMEM_SHARED((_NP, _HH), jnp.float32),
            pltpu.SemaphoreType.DMA,
            pltpu.SemaphoreType.DMA,
        ],
    )
    def agg_k(h0_hbm, h1_hbm, src_hbm, dst_hbm, ew_hbm, out0, out1,
              srca, dstv0, ewv0, rows0, dstv1, ewv1, rows1, acc, gs0, gs1):
        c = lax.axis_index("c")
        s = lax.axis_index("s")
        bufs = ((dstv0, ewv0, rows0, gs0), (dstv1, ewv1, rows1, gs1))

        pltpu.sync_copy(src_hbm.at[pl.ds(s * epc, epc)], srca)

        def zero_rows(i, _):
            for cc in range(_HH // 16):
                rows0[i, pl.ds(cc * 16, 16)] = jnp.zeros((16,), jnp.float32)
            return 0

        lax.fori_loop(0, _CH, zero_rows, 0)
        for j in range(_STRIPE // _CH):
            pltpu.sync_copy(rows0, acc.at[pl.ds(s * _STRIPE + j * _CH, _CH), :])
        plsc.subcore_barrier()

        def fire_gather(b, ci):
            dv, wv, rw, sem = bufs[b]
            base = s * epc + ci * _CH
            sv = srca.at[pl.ds(ci * _CH, _CH)]
            pltpu.async_copy(dst_hbm.at[pl.ds(base, _CH)], dv, sem)
            pltpu.async_copy(ew_hbm.at[pl.ds(base, _CH)], wv, sem)

            @pl.when(c == 0)
            def _():
                pltpu.async_copy(h0_hbm.at[sv], rw, sem)

            @pl.when(c == 1)
            def _():
                pltpu.async_copy(h1_hbm.at[sv], rw, sem)

        def wait_gather(b, ci):
            dv, wv, rw, sem = bufs[b]
            base = s * epc + ci * _CH
            sv = srca.at[pl.ds(ci * _CH, _CH)]
            pltpu.make_async_copy(dst_hbm.at[pl.ds(base, _CH)], dv, sem).wait()
            pltpu.make_async_copy(ew_hbm.at[pl.ds(base, _CH)], wv, sem).wait()

            @pl.when(c == 0)
            def _():
                pltpu.make_async_copy(h0_hbm.at[sv], rw, sem).wait()

            @pl.when(c == 1)
            def _():
                pltpu.make_async_copy(h1_hbm.at[sv], rw, sem).wait()

        def scale_scatter(b, ci):
            dv, wv, rw, _ = bufs[b]
            for k in range(_CH // 16):
                wvec = wv[pl.ds(k * 16, 16)]

                def scale(t, _2, wvec=wvec, k=k):
                    w = _splat(wvec, t)
                    for cc in range(_HH // 16):
                        sl = pl.ds(cc * 16, 16)
                        rw[k * 16 + t, sl] = rw[k * 16 + t, sl] * w
                    return 0

                lax.fori_loop(0, 16, scale, 0)
            pltpu.sync_copy(rw, acc.at[dv], add=True)

        fire_gather(0, 0)

        def itbody(j, _):
            i0 = 2 * j
            fire_gather(1, i0 + 1)
            wait_gather(0, i0)
            scale_scatter(0, i0)

            @pl.when(j + 1 < nit)
            def _():
                fire_gather(0, i0 + 2)

            wait_gather(1, i0 + 1)
            scale_scatter(1, i0 + 1)
            return 0

        lax.fori_loop(0, nit, itbody, 0)
        plsc.subcore_barrier()

        @pl.when(c == 0)
        def _():
            pltpu.sync_copy(acc.at[pl.ds(s * _STRIPE, _STRIPE), :],
                            out0.at[pl.ds(s * _STRIPE, _STRIPE), :])

        @pl.when(c == 1)
        def _():
            pltpu.sync_copy(acc.at[pl.ds(s * _STRIPE, _STRIPE), :],
                            out1.at[pl.ds(s * _STRIPE, _STRIPE), :])

    return agg_k(h0, h1, src, dst, ew)


_RB = 256


def _k1_body(x_ref, w_ref, d0_ref, d1_ref, h0_ref, h1_ref, dinv_ref):
    deg = 1.0 + d0_ref[...] + d1_ref[...]
    dinv = lax.rsqrt(deg)
    hmat = jnp.dot(x_ref[...], w_ref[...], preferred_element_type=jnp.float32)
    ht = dinv * hmat
    h0_ref[...] = ht[:, :_HH]
    h1_ref[...] = ht[:, _HH:]
    dinv_ref[...] = dinv


def _k1_call(xp, w1, d0, d1):
    grid = (_NP // _RB,)
    return pl.pallas_call(
        _k1_body,
        grid=grid,
        in_specs=[
            pl.BlockSpec((_RB, 512), lambda i: (i, 0)),
            pl.BlockSpec((512, _H), lambda i: (0, 0)),
            pl.BlockSpec((_RB, 1), lambda i: (i, 0)),
            pl.BlockSpec((_RB, 1), lambda i: (i, 0)),
        ],
        out_specs=[
            pl.BlockSpec((_RB, _HH), lambda i: (i, 0)),
            pl.BlockSpec((_RB, _HH), lambda i: (i, 0)),
            pl.BlockSpec((_RB, 1), lambda i: (i, 0)),
        ],
        out_shape=[
            jax.ShapeDtypeStruct((_NP, _HH), jnp.float32),
            jax.ShapeDtypeStruct((_NP, _HH), jnp.float32),
            jax.ShapeDtypeStruct((_NP, 1), jnp.float32),
        ],
    )(xp, w1, d0, d1)


def _layer_body(a0_ref, a1_ref, h0_ref, h1_ref, dinv_ref, b_ref, w_ref,
                xk_ref, n0_ref, n1_ref):
    dinv = dinv_ref[...]
    agg = jnp.concatenate([a0_ref[...], a1_ref[...]], axis=1)
    htp = jnp.concatenate([h0_ref[...], h1_ref[...]], axis=1)
    xk = jnp.maximum(dinv * (agg + htp) + b_ref[...], 0.0)
    hn = jnp.dot(xk, w_ref[...], preferred_element_type=jnp.float32)
    htn = dinv * hn
    xk_ref[...] = xk
    n0_ref[...] = htn[:, :_HH]
    n1_ref[...] = htn[:, _HH:]


def _layer_call(a0, a1, h0, h1, dinv, brow, w2):
    grid = (_NP // _RB,)
    half = pl.BlockSpec((_RB, _HH), lambda i: (i, 0))
    return pl.pallas_call(
        _layer_body,
        grid=grid,
        in_specs=[
            half, half, half, half,
            pl.BlockSpec((_RB, 1), lambda i: (i, 0)),
            pl.BlockSpec((1, _H), lambda i: (0, 0)),
            pl.BlockSpec((_H, _H), lambda i: (0, 0)),
        ],
        out_specs=[pl.BlockSpec((_RB, _H), lambda i: (i, 0)), half, half],
        out_shape=[
            jax.ShapeDtypeStruct((_NP, _H), jnp.float32),
            jax.ShapeDtypeStruct((_NP, _HH), jnp.float32),
            jax.ShapeDtypeStruct((_NP, _HH), jnp.float32),
        ],
    )(a0, a1, h0, h1, dinv, brow, w2)


def _x3_body(a0_ref, a1_ref, h0_ref, h1_ref, dinv_ref, b_ref, x1_ref, x2_ref,
             p_ref, x3_ref, sc_ref):
    dinv = dinv_ref[...]
    agg = jnp.concatenate([a0_ref[...], a1_ref[...]], axis=1)
    htp = jnp.concatenate([h0_ref[...], h1_ref[...]], axis=1)
    x3 = jnp.maximum(dinv * (agg + htp) + b_ref[...], 0.0)
    cat = jnp.concatenate([x1_ref[...], x2_ref[...], x3], axis=1)
    p2 = p_ref[...]
    pn = jnp.sqrt(jnp.sum(p2 * p2))
    sco = jax.nn.sigmoid(
        jnp.dot(cat, p2, preferred_element_type=jnp.float32) / pn)
    x3_ref[...] = x3
    sc_ref[...] = sco


def _x3_call(a0, a1, h0, h1, dinv, brow, x1, x2, pcol):
    grid = (_NP // _RB,)
    half = pl.BlockSpec((_RB, _HH), lambda i: (i, 0))
    return pl.pallas_call(
        _x3_body,
        grid=grid,
        in_specs=[
            half, half, half, half,
            pl.BlockSpec((_RB, 1), lambda i: (i, 0)),
            pl.BlockSpec((1, _H), lambda i: (0, 0)),
            pl.BlockSpec((_RB, _H), lambda i: (i, 0)),
            pl.BlockSpec((_RB, _H), lambda i: (i, 0)),
            pl.BlockSpec((3 * _H, 1), lambda i: (0, 0)),
        ],
        out_specs=[
            pl.BlockSpec((_RB, _H), lambda i: (i, 0)),
            pl.BlockSpec((_RB, 1), lambda i: (i, 0)),
        ],
        out_shape=[
            jax.ShapeDtypeStruct((_NP, _H), jnp.float32),
            jax.ShapeDtypeStruct((_NP, 1), jnp.float32),
        ],
    )(a0, a1, h0, h1, dinv, brow, x1, x2, pcol)


_JC = 1024


def _rank_body(srow_ref, brow_ref, scol_ref, bcol_ref, keep_ref, cnt_ref):
    pid = pl.program_id(0)
    srow = srow_ref[...]
    brow = brow_ref[...]
    si = scol_ref[...]
    bi = bcol_ref[...]
    ii = lax.broadcasted_iota(jnp.int32, (_RB, 1), 0) + pid * _RB
    gmin = jnp.min(bi)
    gmax = jnp.max(bi)

    cnt_ref[...] = jnp.zeros((_RB, 1), jnp.float32)
    cg = jnp.zeros((_G, 1), jnp.float32)
    gcol = lax.broadcasted_iota(jnp.int32, (_G, 1), 0)
    for jc in range(_NP // _JC):
        bj = lax.slice(brow, (0, jc * _JC), (1, (jc + 1) * _JC))
        cg = cg + jnp.sum((bj == gcol).astype(jnp.float32), axis=1,
                          keepdims=True)
        sj = lax.slice(srow, (0, jc * _JC), (1, (jc + 1) * _JC))
        ij = lax.broadcasted_iota(jnp.int32, (1, _JC), 1) + jc * _JC
        cmin = jnp.min(bj)
        cmax = jnp.max(bj)

        @pl.when((cmax >= gmin) & (cmin <= gmax))
        def _(bj=bj, sj=sj, ij=ij):
            same = bj == bi
            gt = (sj > si) | ((sj == si) & (ij < ii))
            cnt_ref[...] += jnp.sum((same & gt).astype(jnp.float32), axis=1,
                                    keepdims=True)

    kvec = jnp.ceil(0.8 * cg)
    oh = (bi == gcol.reshape(1, _G)).astype(jnp.float32)
    kati = jnp.dot(oh, kvec, preferred_element_type=jnp.float32)
    keep_ref[...] = (cnt_ref[...] < kati).astype(jnp.float32)


def _rank_call(srow, brow, scol, bcol):
    grid = (_NP // _RB,)
    return pl.pallas_call(
        _rank_body,
        grid=grid,
        in_specs=[
            pl.BlockSpec((1, _NP), lambda i: (0, 0)),
            pl.BlockSpec((1, _NP), lambda i: (0, 0)),
            pl.BlockSpec((_RB, 1), lambda i: (i, 0)),
            pl.BlockSpec((_RB, 1), lambda i: (i, 0)),
        ],
        out_specs=pl.BlockSpec((_RB, 1), lambda i: (i, 0)),
        out_shape=jax.ShapeDtypeStruct((_NP, 1), jnp.float32),
        scratch_shapes=[pltpu.VMEM((_RB, 1), jnp.float32)],
    )(srow, brow, scol, bcol)


_PB = 1024


def _pool_body(x1_ref, x2_ref, x3_ref, sc_ref, kp_ref, bc_ref,
               sum_ref, max_ref, cnt_ref):
    pid = pl.program_id(0)

    @pl.when(pid == 0)
    def _():
        sum_ref[...] = jnp.zeros((_G, 3 * _H), jnp.float32)
        max_ref[...] = jnp.full((_G, 3 * _H), -jnp.inf, jnp.float32)
        cnt_ref[...] = jnp.zeros((_G, 1), jnp.float32)

    cat = jnp.concatenate([x1_ref[...], x2_ref[...], x3_ref[...]], axis=1)
    xp = cat * sc_ref[...]
    kf = kp_ref[...]
    b = bc_ref[...]
    grow = lax.broadcasted_iota(jnp.int32, (1, _G), 1)
    oh = ((b == grow) & (kf > 0.5)).astype(jnp.float32)
    sum_ref[...] += lax.dot_general(oh, xp, (((0,), (0,)), ((), ())),
                                    preferred_element_type=jnp.float32)
    cnt_ref[...] += jnp.sum(oh, axis=0)[:, None]
    gmn = jnp.min(b)
    gmx = jnp.max(b)
    for g in range(_G):
        @pl.when((g >= gmn) & (g <= gmx))
        def _(g=g):
            m = (b == g) & (kf > 0.5)
            mx = jnp.max(jnp.where(m, xp, -jnp.inf), axis=0)
            max_ref[g, :] = jnp.maximum(max_ref[g, :], mx)


def _pool_call(x1, x2, x3, score, keep, bcol):
    grid = (_NP // _PB,)
    full = pl.BlockSpec((_PB, _H), lambda i: (i, 0))
    one = pl.BlockSpec((_PB, 1), lambda i: (i, 0))
    return pl.pallas_call(
        _pool_body,
        grid=grid,
        in_specs=[full, full, full, one, one, one],
        out_specs=[
            pl.BlockSpec((_G, 3 * _H), lambda i: (0, 0)),
            pl.BlockSpec((_G, 3 * _H), lambda i: (0, 0)),
            pl.BlockSpec((_G, 1), lambda i: (0, 0)),
        ],
        out_shape=[
            jax.ShapeDtypeStruct((_G, 3 * _H), jnp.float32),
            jax.ShapeDtypeStruct((_G, 3 * _H), jnp.float32),
            jax.ShapeDtypeStruct((_G, 1), jnp.float32),
        ],
    )(x1, x2, x3, score, keep, bcol)


def _mlp_body(sum_ref, max_ref, cnt_ref, w1_ref, b1_ref, w2_ref, b2_ref,
              w3_ref, b3_ref, out_ref):
    cnt = cnt_ref[...]
    gap = sum_ref[...] / jnp.maximum(cnt, 1.0)
    mx = max_ref[...]
    gmp = jnp.where(jnp.isfinite(mx), mx, 0.0)
    ro = jnp.concatenate([gap, gmp], axis=1)
    h = jnp.maximum(
        jnp.dot(ro, w1_ref[...], preferred_element_type=jnp.float32)
        + b1_ref[...], 0.0)
    h = jnp.maximum(
        jnp.dot(h, w2_ref[...], preferred_element_type=jnp.float32)
        + b2_ref[...], 0.0)
    out_ref[...] = (
        jnp.dot(h, w3_ref[...], preferred_element_type=jnp.float32)
        + b3_ref[...])


def _mlp_call(sums, maxs, cnts, wl1, bl1, wl2, bl2, wl3, bl3):
    return pl.pallas_call(
        _mlp_body,
        out_shape=jax.ShapeDtypeStruct((_G, 3), jnp.float32),
    )(sums, maxs, cnts, wl1, bl1.reshape(1, -1), wl2, bl2.reshape(1, -1),
      wl3, bl3.reshape(1, -1))


def kernel(x, edge_index, edge_attr, batch, W1, b1, W2, b2, p,
           Wl1, bl1, Wl2, bl2, Wl3, bl3):
    src = edge_index[0].astype(jnp.int32)
    dst = edge_index[1].astype(jnp.int32)
    ew = edge_attr.astype(jnp.float32)
    pad_e = _EP - _E
    src = jnp.pad(src, (0, pad_e))
    dst = jnp.pad(dst, (0, pad_e))
    ew = jnp.pad(ew, (0, pad_e))

    xp = jnp.pad(x, ((0, _NP - _N), (0, 0)))
    bcol = jnp.pad(batch.astype(jnp.int32), (0, _NP - _N),
                   constant_values=_G).reshape(_NP, 1)

    dp0, dp1 = _sc_deg_call(dst, ew)
    d0 = dp0[:, 0:1]
    d1 = dp1[:, 0:1]

    h0, h1, dinv = _k1_call(xp, W1, d0, d1)

    b1row = b1.reshape(1, _H)
    b2row = b2.reshape(1, _H)

    a0, a1 = _sc_agg_call(h0, h1, src, dst, ew)
    x1, h0, h1 = _layer_call(a0, a1, h0, h1, dinv, b1row, W2)
    a0, a1 = _sc_agg_call(h0, h1, src, dst, ew)
    x2, h0, h1 = _layer_call(a0, a1, h0, h1, dinv, b2row, W2)
    a0, a1 = _sc_agg_call(h0, h1, src, dst, ew)
    x3, score = _x3_call(a0, a1, h0, h1, dinv, b2row, x1, x2,
                         p.reshape(3 * _H, 1))

    srow = score.reshape(1, _NP)
    brow = bcol.reshape(1, _NP)
    keep = _rank_call(srow, brow, score, bcol)

    sums, maxs, cnts = _pool_call(x1, x2, x3, score, keep, bcol)
    return _mlp_call(sums, maxs, cnts, Wl1, bl1, Wl2, bl2, Wl3, bl3)
